# baseline (speedup 1.0000x reference)
import functools

import jax
import jax.numpy as jnp
from jax.experimental import pallas as pl
from jax.experimental.pallas import tpu as pltpu

_E = 8
_TOPK = 2


def _moe_body(x_ref, gate_w_ref, gp_ref, up_ref, dp_ref,
              out_ref, logits_ref, coef_ref):
    e = pl.program_id(0)
    M = x_ref.shape[0]

    @pl.when(e == 0)
    def _routing():
        x = x_ref[...]
        logits = jax.lax.dot_general(
            x, gate_w_ref[...],
            (((1,), (1,)), ((), ())),
            preferred_element_type=jnp.float32)
        logits_ref[...] = logits
        m = jnp.max(logits, axis=1, keepdims=True)
        p = jnp.exp(logits - m)
        p = p / jnp.sum(p, axis=1, keepdims=True)
        lane = jax.lax.broadcasted_iota(jnp.int32, p.shape, 1)
        e1 = jnp.argmax(p, axis=1)[:, None]
        mask1 = lane == e1
        p_masked = jnp.where(mask1, -jnp.inf, p)
        e2 = jnp.argmax(p_masked, axis=1)[:, None]
        mask2 = lane == e2
        m1 = jnp.max(p, axis=1, keepdims=True)
        m2 = jnp.max(p_masked, axis=1, keepdims=True)
        denom = m1 + m2
        coef_ref[...] = jnp.where(mask1 | mask2, p, 0.0) / denom

    x = x_ref[...].astype(jnp.bfloat16)
    g = jax.lax.dot_general(
        x, gp_ref[0], (((1,), (1,)), ((), ())),
        preferred_element_type=jnp.float32)
    u = jax.lax.dot_general(
        x, up_ref[0], (((1,), (1,)), ((), ())),
        preferred_element_type=jnp.float32)
    h = ((g * jax.nn.sigmoid(g)) * u).astype(jnp.bfloat16)
    y = jax.lax.dot_general(
        h, dp_ref[0], (((1,), (1,)), ((), ())),
        preferred_element_type=jnp.float32)
    c = coef_ref[...]
    lane = jax.lax.broadcasted_iota(jnp.int32, c.shape, 1)
    coef = jnp.sum(jnp.where(lane == e, c, 0.0), axis=1, keepdims=True)
    contrib = coef * y

    @pl.when(e == 0)
    def _init():
        out_ref[...] = contrib

    @pl.when(e != 0)
    def _acc():
        out_ref[...] += contrib


@functools.partial(jax.jit, static_argnames=())
def kernel(hidden_states, gate_w, gate_proj_w, up_proj_w, down_proj_w):
    B, S, H = hidden_states.shape
    M = B * S
    E, FF, _ = gate_proj_w.shape
    x = hidden_states.reshape(M, H)
    gp = gate_proj_w.astype(jnp.bfloat16)
    up = up_proj_w.astype(jnp.bfloat16)
    dp = down_proj_w.astype(jnp.bfloat16)

    out, logits = pl.pallas_call(
        _moe_body,
        grid=(E,),
        in_specs=[
            pl.BlockSpec((M, H), lambda e: (0, 0)),
            pl.BlockSpec((E, H), lambda e: (0, 0)),
            pl.BlockSpec((1, FF, H), lambda e: (e, 0, 0)),
            pl.BlockSpec((1, FF, H), lambda e: (e, 0, 0)),
            pl.BlockSpec((1, H, FF), lambda e: (e, 0, 0)),
        ],
        out_specs=[
            pl.BlockSpec((M, H), lambda e: (0, 0)),
            pl.BlockSpec((M, E), lambda e: (0, 0)),
        ],
        out_shape=[
            jax.ShapeDtypeStruct((M, H), jnp.float32),
            jax.ShapeDtypeStruct((M, E), jnp.float32),
        ],
        scratch_shapes=[pltpu.VMEM((M, E), jnp.float32)],
        compiler_params=pltpu.CompilerParams(
            dimension_semantics=("arbitrary",),
        ),
    )(x, gate_w, gp, up, dp)

    return out.reshape(B, S, H), logits


# speedup vs baseline: 1.0257x; 1.0257x over previous
import functools

import jax
import jax.numpy as jnp
from jax import lax
from jax.experimental import pallas as pl
from jax.experimental.pallas import tpu as pltpu
from jax.experimental.pallas import tpu_sc as plsc

_E = 8
_TOPK = 2
_BLK = 512
_NBLK = 15
_NW = 32
_L = 16


def _routing_body(x_ref, gw_ref, logits_ref, e1_ref, e2_ref, w1_ref, w2_ref):
    x = x_ref[...]
    logits = lax.dot_general(
        x, gw_ref[...], (((1,), (1,)), ((), ())),
        preferred_element_type=jnp.float32)
    logits_ref[...] = logits
    m = jnp.max(logits, axis=1, keepdims=True)
    p = jnp.exp(logits - m)
    p = p / jnp.sum(p, axis=1, keepdims=True)
    lane = lax.broadcasted_iota(jnp.int32, p.shape, 1)
    e1 = jnp.argmax(p, axis=1)[:, None]
    mask1 = lane == e1
    p_masked = jnp.where(mask1, -jnp.inf, p)
    e2 = jnp.argmax(p_masked, axis=1)[:, None]
    m1 = jnp.max(p, axis=1, keepdims=True)
    m2 = jnp.max(p_masked, axis=1, keepdims=True)
    denom = m1 + m2
    e1_ref[...] = e1
    e2_ref[...] = e2
    w1_ref[...] = m1 / denom
    w2_ref[...] = m2 / denom


def _routing(x, gate_w):
    M, H = x.shape
    E = gate_w.shape[0]
    return pl.pallas_call(
        _routing_body,
        out_shape=[
            jax.ShapeDtypeStruct((M, E), jnp.float32),
            jax.ShapeDtypeStruct((M, 1), jnp.int32),
            jax.ShapeDtypeStruct((M, 1), jnp.int32),
            jax.ShapeDtypeStruct((M, 1), jnp.float32),
            jax.ShapeDtypeStruct((M, 1), jnp.float32),
        ],
    )(x, gate_w)


def _sort_scatter_body(sel_hbm, x_hbm,
                       pos_hbm, xs_hbm, bexp_hbm,
                       keys_v, pos_v, rows_v, bexp_v, sem, sem2):
    wid = lax.axis_index("c") * 16 + lax.axis_index("s")
    nchunk_per_tile = 8

    tok0 = (wid % 16) * 128
    rows_dma = pltpu.async_copy(x_hbm.at[pl.ds(tok0, 128)], rows_v, sem)
    pltpu.sync_copy(sel_hbm, keys_v)

    def count_span(lo, hi):
        def body(j, accs):
            kva = keys_v[pl.ds(j * 2 * _L, _L)]
            kvb = keys_v[pl.ds((j * 2 + 1) * _L, _L)]
            return tuple(acc + jnp.where(kva == e, 1, 0)
                         + jnp.where(kvb == e, 1, 0)
                         for e, acc in enumerate(accs))
        init = tuple(jnp.zeros((_L,), jnp.int32) for _ in range(_E))
        return lax.fori_loop(lo, hi, body, init)

    c0 = wid * nchunk_per_tile
    pre = count_span(0, c0 // 2)
    own = count_span(c0 // 2, c0 // 2 + nchunk_per_tile // 2)
    rest = count_span(c0 // 2 + nchunk_per_tile // 2, 128)

    def vec_total(acc):
        s = acc[0]
        for l in range(1, _L):
            s = s + acc[l]
        return s

    base = []
    cumblocks = []
    off = jnp.int32(0)
    cum = jnp.int32(0)
    for e in range(_E):
        pre_e = vec_total(pre[e])
        n_e = pre_e + vec_total(own[e]) + vec_total(rest[e])
        nblk_e = (n_e + (_BLK - 1)) // _BLK
        base.append(off + pre_e)
        off = off + nblk_e * _BLK
        cum = cum + nblk_e
        cumblocks.append(cum)

    iota = lax.iota(jnp.int32, _L)
    cur = list(base)
    one = jnp.int32(1)
    zero = jnp.int32(0)
    for c in range(nchunk_per_tile):
        kv = keys_v[pl.ds((c0 + c) * _L, _L)]
        posc = jnp.zeros((_L,), jnp.int32)
        for l in range(_L):
            k = kv[l]
            p_l = cur[_E - 1]
            for e in range(_E - 2, -1, -1):
                p_l = jnp.where(k == e, cur[e], p_l)
            posc = jnp.where(iota == l, p_l, posc)
            for e in range(_E):
                cur[e] = cur[e] + jnp.where(k == e, one, zero)
        pos_v[pl.ds(c * _L, _L)] = posc

    rows_dma.wait()
    scat = pltpu.async_copy(rows_v, xs_hbm.at[pos_v], sem2)
    pltpu.sync_copy(pos_v, pos_hbm.at[pl.ds(wid * 128, 128)])
    scat.wait()

    @pl.when(wid == 0)
    def _():
        for half in range(2):
            bv = iota + half * _L
            ev = jnp.zeros((_L,), jnp.int32)
            for e in range(_E):
                ev = ev + jnp.where(bv >= cumblocks[e], 1, 0)
            bexp_v[pl.ds(half * _L, _L)] = ev
        pltpu.sync_copy(bexp_v, bexp_hbm)


def _sort_scatter(sel, x):
    M, H = x.shape
    P = _NBLK * _BLK
    mesh = plsc.VectorSubcoreMesh(core_axis_name="c", subcore_axis_name="s")
    f = pl.kernel(
        _sort_scatter_body,
        out_type=[
            jax.ShapeDtypeStruct((2 * M,), jnp.int32),
            jax.ShapeDtypeStruct((P, H), jnp.float32),
            jax.ShapeDtypeStruct((2 * _L,), jnp.int32),
        ],
        mesh=mesh,
        scratch_types=[
            pltpu.VMEM((2 * M,), jnp.int32),
            pltpu.VMEM((128,), jnp.int32),
            pltpu.VMEM((128, H), jnp.float32),
            pltpu.VMEM((2 * _L,), jnp.int32),
            pltpu.SemaphoreType.DMA,
            pltpu.SemaphoreType.DMA,
        ],
    )
    return f(sel, x)


def _ffn_body(bexp_sref, xs_ref, gp_ref, up_ref, dp_ref, y_ref):
    i = pl.program_id(0)

    @pl.when(bexp_sref[i] < _E)
    def _():
        xb = xs_ref[...]
        g = lax.dot_general(
            xb, gp_ref[0], (((1,), (1,)), ((), ())),
            preferred_element_type=jnp.float32)
        u = lax.dot_general(
            xb, up_ref[0], (((1,), (1,)), ((), ())),
            preferred_element_type=jnp.float32)
        h = (g * jax.nn.sigmoid(g)) * u
        y_ref[...] = lax.dot_general(
            h, dp_ref[0], (((1,), (1,)), ((), ())),
            preferred_element_type=jnp.float32)


def _grouped_ffn(bexp, xs, gate_proj_w, up_proj_w, down_proj_w):
    P, H = xs.shape
    E, FF, _ = gate_proj_w.shape

    def wmap(i, be):
        return (jnp.minimum(be[i], E - 1), 0, 0)

    grid_spec = pltpu.PrefetchScalarGridSpec(
        num_scalar_prefetch=1,
        grid=(_NBLK,),
        in_specs=[
            pl.BlockSpec((_BLK, H),
                         lambda i, be: (jnp.where(be[i] < _E, i, 0), 0)),
            pl.BlockSpec((1, FF, H), wmap),
            pl.BlockSpec((1, FF, H), wmap),
            pl.BlockSpec((1, H, FF), wmap),
        ],
        out_specs=pl.BlockSpec((_BLK, H), lambda i, be: (i, 0)),
    )
    return pl.pallas_call(
        _ffn_body,
        grid_spec=grid_spec,
        out_shape=jax.ShapeDtypeStruct((P, H), jnp.float32),
        compiler_params=pltpu.CompilerParams(
            dimension_semantics=("arbitrary",),
        ),
    )(bexp, xs, gate_proj_w, up_proj_w, down_proj_w)


def _gather2_body(y_hbm, pos_hbm, y1_hbm, y2_hbm, idx1_v, idx2_v,
                  buf1, buf2, sem, sem2):
    wid = lax.axis_index("c") * 16 + lax.axis_index("s")
    M = y1_hbm.shape[0]
    tpt = M // _NW
    t0 = wid * tpt

    pltpu.sync_copy(pos_hbm.at[pl.ds(t0, tpt)], idx1_v)
    pltpu.sync_copy(pos_hbm.at[pl.ds(M + t0, tpt)], idx2_v)
    g1 = pltpu.async_copy(y_hbm.at[idx1_v], buf1, sem)
    g2 = pltpu.async_copy(y_hbm.at[idx2_v], buf2, sem2)
    g1.wait()
    pltpu.sync_copy(buf1, y1_hbm.at[pl.ds(t0, tpt)])
    g2.wait()
    pltpu.sync_copy(buf2, y2_hbm.at[pl.ds(t0, tpt)])


def _gather2(y, pos, M):
    H = y.shape[1]
    tpt = M // _NW
    mesh = plsc.VectorSubcoreMesh(core_axis_name="c", subcore_axis_name="s")
    f = pl.kernel(
        _gather2_body,
        out_type=[
            jax.ShapeDtypeStruct((M, H), jnp.float32),
            jax.ShapeDtypeStruct((M, H), jnp.float32),
        ],
        mesh=mesh,
        scratch_types=[
            pltpu.VMEM((tpt,), jnp.int32),
            pltpu.VMEM((tpt,), jnp.int32),
            pltpu.VMEM((tpt, H), jnp.float32),
            pltpu.VMEM((tpt, H), jnp.float32),
            pltpu.SemaphoreType.DMA,
            pltpu.SemaphoreType.DMA,
        ],
    )
    return f(y, pos)


def _wsum_body(y1_ref, y2_ref, w1_ref, w2_ref, out_ref):
    out_ref[...] = y1_ref[...] * w1_ref[...] + y2_ref[...] * w2_ref[...]


def _wsum(y1, y2, w1, w2):
    M, H = y1.shape
    return pl.pallas_call(
        _wsum_body,
        grid=(4,),
        in_specs=[
            pl.BlockSpec((M // 4, H), lambda i: (i, 0)),
            pl.BlockSpec((M // 4, H), lambda i: (i, 0)),
            pl.BlockSpec((M // 4, 1), lambda i: (i, 0)),
            pl.BlockSpec((M // 4, 1), lambda i: (i, 0)),
        ],
        out_specs=pl.BlockSpec((M // 4, H), lambda i: (i, 0)),
        out_shape=jax.ShapeDtypeStruct((M, H), jnp.float32),
    )(y1, y2, w1, w2)


@functools.partial(jax.jit, static_argnames=())
def kernel(hidden_states, gate_w, gate_proj_w, up_proj_w, down_proj_w):
    B, S, H = hidden_states.shape
    M = B * S
    x = hidden_states.reshape(M, H)

    logits, e1, e2, w1, w2 = _routing(x, gate_w)
    sel = jnp.concatenate([e1.reshape(M), e2.reshape(M)])
    pos, xs, bexp = _sort_scatter(sel, x)
    y = _grouped_ffn(bexp, xs, gate_proj_w, up_proj_w, down_proj_w)
    y1, y2 = _gather2(y, pos, M)
    out = _wsum(y1, y2, w1, w2)
    return out.reshape(B, S, H), logits
